# baseline (speedup 1.0000x reference)
import functools

import jax
import jax.numpy as jnp
from jax import lax
from jax.experimental import pallas as pl
from jax.experimental.pallas import tpu as pltpu
from jax.experimental.pallas import tpu_sc as plsc

H2 = 512
ACT_V, ACT_D, ACT_F = 100000, 64, 5
PER_V, PER_D, PER_F = 1000, 32, 23
HH_V, HH_D, HH_F = 1000, 32, 9
T, N, H = 50, 4096, 8
R_TOTAL = T + 2 * H + 2

NC, NS = 2, 16
NW = NC * NS

NSPLIT = 2
TH = T // NSPLIT
MA = TH * N
MH = H * N
PW_A, PW_P, PW_H = MA // NW, N // NW, MH // NW
CA = 1600


def _sc_gather_small_body(per_tab, per_idx, hh_tab, hh_idx,
                          per_out, hh_out,
                          idx_p, rows_p, idx_h, rows_h, sem):
    wid = lax.axis_index("s") * NC + lax.axis_index("c")

    for f in range(PER_F):
        m = wid * PW_P
        pltpu.sync_copy(per_idx.at[pl.ds(f * N + m, PW_P)], idx_p)
        pltpu.async_copy(per_tab.at[idx_p], rows_p, sem).wait()
        pltpu.sync_copy(rows_p,
                        per_out.at[pl.ds(m, PW_P), pl.ds(f * PER_D, PER_D)])

    for f in range(HH_F):
        m = wid * PW_H
        pltpu.sync_copy(hh_idx.at[pl.ds(f * MH + m, PW_H)], idx_h)
        pltpu.async_copy(hh_tab.at[idx_h], rows_h, sem).wait()
        pltpu.sync_copy(rows_h,
                        hh_out.at[pl.ds(m, PW_H), pl.ds(f * HH_D, HH_D)])


def _sc_gather_act_body(act_tab, act_idx, act_out, idx_a, rows_a, sem):
    wid = lax.axis_index("s") * NC + lax.axis_index("c")

    for f in range(ACT_F):
        base = wid * PW_A

        def body_a(i, carry, f=f, base=base):
            m = base + i * CA
            pltpu.sync_copy(act_idx.at[pl.ds(f * MA + m, CA)], idx_a)
            pltpu.async_copy(act_tab.at[idx_a], rows_a, sem).wait()
            pltpu.sync_copy(rows_a,
                            act_out.at[pl.ds(m, CA), pl.ds(f * ACT_D, ACT_D)])
            return carry

        lax.fori_loop(0, PW_A // CA, body_a, 0)


_sc_mesh = plsc.VectorSubcoreMesh(core_axis_name="c", subcore_axis_name="s")

_sc_gather_small = functools.partial(
    pl.kernel,
    mesh=_sc_mesh,
    out_type=[
        jax.ShapeDtypeStruct((N, PER_F * PER_D), jnp.bfloat16),
        jax.ShapeDtypeStruct((MH, HH_F * HH_D), jnp.bfloat16),
    ],
    scratch_types=[
        pltpu.VMEM((PW_P,), jnp.int32),
        pltpu.VMEM((PW_P, PER_D), jnp.bfloat16),
        pltpu.VMEM((PW_H,), jnp.int32),
        pltpu.VMEM((PW_H, HH_D), jnp.bfloat16),
        pltpu.SemaphoreType.DMA,
    ],
    compiler_params=pltpu.CompilerParams(use_tc_tiling_on_sc=False),
)(_sc_gather_small_body)

_sc_gather_act = functools.partial(
    pl.kernel,
    mesh=_sc_mesh,
    out_type=jax.ShapeDtypeStruct((MA, ACT_F * ACT_D), jnp.bfloat16),
    scratch_types=[
        pltpu.VMEM((CA,), jnp.int32),
        pltpu.VMEM((CA, ACT_D), jnp.bfloat16),
        pltpu.SemaphoreType.DMA,
    ],
    compiler_params=pltpu.CompilerParams(use_tc_tiling_on_sc=False),
)(_sc_gather_act_body)


BN = 2048
NJ = N // BN
R_SMALL = 2 * H + 2


def _head_body(perg, hhg, p_w, p_b, h_w, h_b, sep_r, out):
    r = pl.program_id(0)
    is_per = r == 0
    is_hh = jnp.logical_and(r >= 2, lax.rem(r, 2) == 0)
    is_sep = jnp.logical_and(jnp.logical_not(is_per), jnp.logical_not(is_hh))

    @pl.when(is_per)
    def _():
        out[0] = jnp.dot(perg[...], p_w[...],
                         preferred_element_type=jnp.float32) + p_b[...]

    @pl.when(is_hh)
    def _():
        out[0] = jnp.dot(hhg[0], h_w[...],
                         preferred_element_type=jnp.float32) + h_b[...]

    @pl.when(is_sep)
    def _():
        out[0] = jnp.broadcast_to(sep_r[...], (BN, H2))


def _act_body(actg, a_w, a_b, alias, out):
    del alias
    out[0] = jnp.dot(actg[0], a_w[...],
                     preferred_element_type=jnp.float32) + a_b[...]


def _project_head(per_g, hh_g, person_W, person_b, hh_W, hh_b, sep):
    zz = lambda r, j: (0, 0)
    hh_pred = lambda r: jnp.logical_and(r >= 2, lax.rem(r, 2) == 0)
    return pl.pallas_call(
        _head_body,
        grid=(R_SMALL, NJ),
        in_specs=[
            pl.BlockSpec((BN, PER_F * PER_D),
                         lambda r, j: (jnp.where(r == 0, j, 0), 0)),
            pl.BlockSpec((1, BN, HH_F * HH_D),
                         lambda r, j: (jnp.where(hh_pred(r), (r - 2) // 2, 0),
                                       jnp.where(hh_pred(r), j, 0), 0)),
            pl.BlockSpec((PER_F * PER_D, H2), zz),
            pl.BlockSpec((1, H2), zz),
            pl.BlockSpec((HH_F * HH_D, H2), zz),
            pl.BlockSpec((1, H2), zz),
            pl.BlockSpec((1, H2), zz),
        ],
        out_specs=pl.BlockSpec((1, BN, H2), lambda r, j: (r, j, 0)),
        out_shape=jax.ShapeDtypeStruct((R_TOTAL, N, H2), jnp.float32),
    )(per_g, hh_g, person_W, person_b.reshape(1, H2), hh_W,
      hh_b.reshape(1, H2), sep.reshape(1, H2))


def _project_act(act_g, act_W, act_b, buf, half):
    zz = lambda r, j: (0, 0)
    base = R_SMALL + half * TH
    return pl.pallas_call(
        _act_body,
        grid=(TH, NJ),
        in_specs=[
            pl.BlockSpec((1, BN, ACT_F * ACT_D), lambda r, j: (r, j, 0)),
            pl.BlockSpec((ACT_F * ACT_D, H2), zz),
            pl.BlockSpec((1, H2), zz),
            pl.BlockSpec(memory_space=pl.ANY),
        ],
        out_specs=pl.BlockSpec((1, BN, H2), lambda r, j: (r + base, j, 0)),
        out_shape=jax.ShapeDtypeStruct((R_TOTAL, N, H2), jnp.float32),
        input_output_aliases={3: 0},
    )(act_g, act_W, act_b.reshape(1, H2), buf)


def kernel(activity_chain, target_person, household_members, act_tables,
           person_tables, hh_tables, act_W, act_b, person_W, person_b,
           hh_W, hh_b, sep):
    act_idx = [
        (activity_chain[h * TH:(h + 1) * TH].transpose(2, 0, 1)
         .reshape(ACT_F, MA)
         + jnp.arange(ACT_F, dtype=jnp.int32)[:, None] * ACT_V).reshape(-1)
        for h in range(NSPLIT)
    ]
    per_idx = (target_person.transpose(2, 0, 1).reshape(PER_F, N)
               + jnp.arange(PER_F, dtype=jnp.int32)[:, None] * PER_V).reshape(-1)
    hh_idx = (household_members.transpose(2, 0, 1).reshape(HH_F, MH)
              + jnp.arange(HH_F, dtype=jnp.int32)[:, None] * HH_V).reshape(-1)

    per_g, hh_g = _sc_gather_small(
        person_tables.astype(jnp.bfloat16).reshape(PER_F * PER_V, PER_D),
        per_idx,
        hh_tables.astype(jnp.bfloat16).reshape(HH_F * HH_V, HH_D), hh_idx)

    act_tab = act_tables.astype(jnp.bfloat16).reshape(ACT_F * ACT_V, ACT_D)
    act_g = [_sc_gather_act(act_tab, act_idx[h]) for h in range(NSPLIT)]

    buf = _project_head(per_g, hh_g.reshape(H, N, HH_F * HH_D),
                        person_W.astype(jnp.bfloat16), person_b,
                        hh_W.astype(jnp.bfloat16), hh_b, sep)

    act_W16 = act_W.astype(jnp.bfloat16)
    for h in range(NSPLIT):
        buf = _project_act(act_g[h].reshape(TH, N, ACT_F * ACT_D),
                           act_W16, act_b, buf, h)
    return buf


# speedup vs baseline: 1.2826x; 1.2826x over previous
import functools

import jax
import jax.numpy as jnp
from jax import lax
from jax.experimental import pallas as pl
from jax.experimental.pallas import tpu as pltpu
from jax.experimental.pallas import tpu_sc as plsc

H2 = 512
ACT_V, ACT_D, ACT_F = 100000, 64, 5
PER_V, PER_D, PER_F = 1000, 32, 23
HH_V, HH_D, HH_F = 1000, 32, 9
T, N, H = 50, 4096, 8
R_TOTAL = T + 2 * H + 2

KA, KP, KH = 384, 768, 384
DA, DP, DH = ACT_F * ACT_D, PER_F * PER_D, HH_F * HH_D

NC, NS = 2, 16
NW = NC * NS

NSPLIT = 2
TH = T // NSPLIT
MA = TH * N
MH = H * N
PW_A, PW_P, PW_H = MA // NW, N // NW, MH // NW
CA = 800


def _zero_fill(ref, nrow, ncol):
    z = jnp.zeros((16,), jnp.float32)

    def body(i, carry):
        for j in range(ncol // 16):
            ref[i, pl.ds(j * 16, 16)] = z
        return carry

    lax.fori_loop(0, nrow, body, 0)


def _sc_gather_small_body(per_tab, per_idx, hh_tab, hh_idx,
                          per_out, hh_out,
                          idx_p, rows_p, idx_h, rows_h, zrows, sem):
    wid = lax.axis_index("s") * NC + lax.axis_index("c")
    _zero_fill(zrows, PW_H, PER_D)

    for f in range(PER_F):
        m = wid * PW_P
        pltpu.sync_copy(per_idx.at[pl.ds(f * N + m, PW_P)], idx_p)
        pltpu.async_copy(per_tab.at[idx_p], rows_p, sem).wait()
        pltpu.sync_copy(rows_p,
                        per_out.at[pl.ds(m, PW_P), pl.ds(f * PER_D, PER_D)])
    pltpu.sync_copy(zrows.at[pl.ds(0, PW_P)],
                    per_out.at[pl.ds(wid * PW_P, PW_P), pl.ds(DP, PER_D)])

    for f in range(HH_F):
        m = wid * PW_H
        pltpu.sync_copy(hh_idx.at[pl.ds(f * MH + m, PW_H)], idx_h)
        pltpu.async_copy(hh_tab.at[idx_h], rows_h, sem).wait()
        pltpu.sync_copy(rows_h,
                        hh_out.at[pl.ds(m, PW_H), pl.ds(f * HH_D, HH_D)])
    for f in range(HH_F, KH // HH_D):
        pltpu.sync_copy(zrows,
                        hh_out.at[pl.ds(wid * PW_H, PW_H),
                                  pl.ds(f * HH_D, HH_D)])


def _sc_gather_act_body(act_tab, act_idx, act_out, idx_a, rows_a, zrows, sem):
    wid = lax.axis_index("s") * NC + lax.axis_index("c")
    _zero_fill(zrows, CA, ACT_D)

    for f in range(ACT_F):
        base = wid * PW_A

        def body_a(i, carry, f=f, base=base):
            m = base + i * CA
            pltpu.sync_copy(act_idx.at[pl.ds(f * MA + m, CA)], idx_a)
            pltpu.async_copy(act_tab.at[idx_a], rows_a, sem).wait()
            pltpu.sync_copy(rows_a,
                            act_out.at[pl.ds(m, CA), pl.ds(f * ACT_D, ACT_D)])
            return carry

        lax.fori_loop(0, PW_A // CA, body_a, 0)

    def body_z(i, carry):
        m = wid * PW_A + i * CA
        pltpu.sync_copy(zrows, act_out.at[pl.ds(m, CA), pl.ds(DA, ACT_D)])
        return carry

    lax.fori_loop(0, PW_A // CA, body_z, 0)


_sc_mesh = plsc.VectorSubcoreMesh(core_axis_name="c", subcore_axis_name="s")

_sc_gather_small = functools.partial(
    pl.kernel,
    mesh=_sc_mesh,
    out_type=[
        jax.ShapeDtypeStruct((N, KP), jnp.float32),
        jax.ShapeDtypeStruct((MH, KH), jnp.float32),
    ],
    scratch_types=[
        pltpu.VMEM((PW_P,), jnp.int32),
        pltpu.VMEM((PW_P, PER_D), jnp.float32),
        pltpu.VMEM((PW_H,), jnp.int32),
        pltpu.VMEM((PW_H, HH_D), jnp.float32),
        pltpu.VMEM((PW_H, HH_D), jnp.float32),
        pltpu.SemaphoreType.DMA,
    ],
    compiler_params=pltpu.CompilerParams(use_tc_tiling_on_sc=False),
)(_sc_gather_small_body)

_sc_gather_act = functools.partial(
    pl.kernel,
    mesh=_sc_mesh,
    out_type=jax.ShapeDtypeStruct((MA, KA), jnp.float32),
    scratch_types=[
        pltpu.VMEM((CA,), jnp.int32),
        pltpu.VMEM((CA, ACT_D), jnp.float32),
        pltpu.VMEM((CA, ACT_D), jnp.float32),
        pltpu.SemaphoreType.DMA,
    ],
    compiler_params=pltpu.CompilerParams(use_tc_tiling_on_sc=False),
)(_sc_gather_act_body)


BN = 2048
NJ = N // BN
R_SMALL = 2 * H + 2


def _head_body(perg, hhg, p_w, p_b, h_w, h_b, sep_r, out):
    r = pl.program_id(0)
    is_per = r == 0
    is_hh = jnp.logical_and(r >= 2, lax.rem(r, 2) == 0)
    is_sep = jnp.logical_and(jnp.logical_not(is_per), jnp.logical_not(is_hh))

    @pl.when(is_per)
    def _():
        out[0] = jnp.dot(perg[...].astype(jnp.bfloat16),
                         p_w[...].astype(jnp.bfloat16),
                         preferred_element_type=jnp.float32) + p_b[...]

    @pl.when(is_hh)
    def _():
        out[0] = jnp.dot(hhg[0].astype(jnp.bfloat16),
                         h_w[...].astype(jnp.bfloat16),
                         preferred_element_type=jnp.float32) + h_b[...]

    @pl.when(is_sep)
    def _():
        out[0] = jnp.broadcast_to(sep_r[...], (BN, H2))


def _act_body(actg, a_w, a_b, alias, out):
    del alias
    out[0] = jnp.dot(actg[0].astype(jnp.bfloat16),
                     a_w[...].astype(jnp.bfloat16),
                     preferred_element_type=jnp.float32) + a_b[...]


def _project_head(per_g, hh_g, person_W, person_b, hh_W, hh_b, sep):
    zz = lambda r, j: (0, 0)
    hh_pred = lambda r: jnp.logical_and(r >= 2, lax.rem(r, 2) == 0)
    return pl.pallas_call(
        _head_body,
        grid=(R_SMALL, NJ),
        in_specs=[
            pl.BlockSpec((BN, KP), lambda r, j: (jnp.where(r == 0, j, 0), 0)),
            pl.BlockSpec((1, BN, KH),
                         lambda r, j: (jnp.where(hh_pred(r), (r - 2) // 2, 0),
                                       jnp.where(hh_pred(r), j, 0), 0)),
            pl.BlockSpec((KP, H2), zz),
            pl.BlockSpec((1, H2), zz),
            pl.BlockSpec((KH, H2), zz),
            pl.BlockSpec((1, H2), zz),
            pl.BlockSpec((1, H2), zz),
        ],
        out_specs=pl.BlockSpec((1, BN, H2), lambda r, j: (r, j, 0)),
        out_shape=jax.ShapeDtypeStruct((R_TOTAL, N, H2), jnp.float32),
    )(per_g, hh_g, person_W, person_b.reshape(1, H2), hh_W,
      hh_b.reshape(1, H2), sep.reshape(1, H2))


def _project_act(act_g, act_W, act_b, buf, half):
    zz = lambda r, j: (0, 0)
    base = R_SMALL + half * TH
    return pl.pallas_call(
        _act_body,
        grid=(TH, NJ),
        in_specs=[
            pl.BlockSpec((1, BN, KA), lambda r, j: (r, j, 0)),
            pl.BlockSpec((KA, H2), zz),
            pl.BlockSpec((1, H2), zz),
            pl.BlockSpec(memory_space=pl.ANY),
        ],
        out_specs=pl.BlockSpec((1, BN, H2), lambda r, j: (r + base, j, 0)),
        out_shape=jax.ShapeDtypeStruct((R_TOTAL, N, H2), jnp.float32),
        input_output_aliases={3: 0},
    )(act_g, act_W, act_b.reshape(1, H2), buf)


def _pad_w(w, k):
    return jnp.concatenate(
        [w, jnp.zeros((k - w.shape[0], w.shape[1]), w.dtype)], axis=0)


def kernel(activity_chain, target_person, household_members, act_tables,
           person_tables, hh_tables, act_W, act_b, person_W, person_b,
           hh_W, hh_b, sep):
    act_idx = [
        (activity_chain[h * TH:(h + 1) * TH].transpose(2, 0, 1)
         .reshape(ACT_F, MA)
         + jnp.arange(ACT_F, dtype=jnp.int32)[:, None] * ACT_V).reshape(-1)
        for h in range(NSPLIT)
    ]
    per_idx = (target_person.transpose(2, 0, 1).reshape(PER_F, N)
               + jnp.arange(PER_F, dtype=jnp.int32)[:, None] * PER_V).reshape(-1)
    hh_idx = (household_members.transpose(2, 0, 1).reshape(HH_F, MH)
              + jnp.arange(HH_F, dtype=jnp.int32)[:, None] * HH_V).reshape(-1)

    per_g, hh_g = _sc_gather_small(
        person_tables.reshape(PER_F * PER_V, PER_D), per_idx,
        hh_tables.reshape(HH_F * HH_V, HH_D), hh_idx)

    act_tab = act_tables.reshape(ACT_F * ACT_V, ACT_D)
    act_g = [_sc_gather_act(act_tab, act_idx[h]) for h in range(NSPLIT)]

    buf = _project_head(per_g, hh_g.reshape(H, N, KH),
                        _pad_w(person_W, KP), person_b,
                        _pad_w(hh_W, KH), hh_b, sep)

    act_Wp = _pad_w(act_W, KA)
    for h in range(NSPLIT):
        buf = _project_act(act_g[h].reshape(TH, N, KA), act_Wp, act_b, buf, h)
    return buf


# speedup vs baseline: 1.3143x; 1.0247x over previous
import functools

import jax
import jax.numpy as jnp
from jax import lax
from jax.experimental import pallas as pl
from jax.experimental.pallas import tpu as pltpu
from jax.experimental.pallas import tpu_sc as plsc

H2 = 512
ACT_V, ACT_D, ACT_F = 100000, 64, 5
PER_V, PER_D, PER_F = 1000, 32, 23
HH_V, HH_D, HH_F = 1000, 32, 9
T, N, H = 50, 4096, 8
R_TOTAL = T + 2 * H + 2

KA, KP, KH = 320, 736, 288
DA, DP, DH = ACT_F * ACT_D, PER_F * PER_D, HH_F * HH_D

NC, NS = 2, 16
NW = NC * NS

NSPLIT = 2
TH = T // NSPLIT
MA = TH * N
MH = H * N
PW_A, PW_P, PW_H = MA // NW, N // NW, MH // NW
CA = 1600


def _sc_gather_small_body(per_tab, per_idx, hh_tab, hh_idx,
                          per_out, hh_out,
                          idx_p, rows_p, idx_h, rows_h, sem):
    wid = lax.axis_index("s") * NC + lax.axis_index("c")

    for f in range(PER_F):
        m = wid * PW_P
        pltpu.sync_copy(per_idx.at[pl.ds(f * N + m, PW_P)], idx_p)
        pltpu.async_copy(per_tab.at[idx_p], rows_p, sem).wait()
        pltpu.sync_copy(rows_p,
                        per_out.at[pl.ds(m, PW_P), pl.ds(f * PER_D, PER_D)])
    for f in range(HH_F):
        m = wid * PW_H
        pltpu.sync_copy(hh_idx.at[pl.ds(f * MH + m, PW_H)], idx_h)
        pltpu.async_copy(hh_tab.at[idx_h], rows_h, sem).wait()
        pltpu.sync_copy(rows_h,
                        hh_out.at[pl.ds(m, PW_H), pl.ds(f * HH_D, HH_D)])


def _sc_gather_act_body(act_tab, act_idx, act_out, idx_a, rows_a, sem):
    wid = lax.axis_index("s") * NC + lax.axis_index("c")

    for f in range(ACT_F):
        base = wid * PW_A

        def body_a(i, carry, f=f, base=base):
            m = base + i * CA
            pltpu.sync_copy(act_idx.at[pl.ds(f * MA + m, CA)], idx_a)
            pltpu.async_copy(act_tab.at[idx_a], rows_a, sem).wait()
            pltpu.sync_copy(rows_a,
                            act_out.at[pl.ds(m, CA), pl.ds(f * ACT_D, ACT_D)])
            return carry

        lax.fori_loop(0, PW_A // CA, body_a, 0)


_sc_mesh = plsc.VectorSubcoreMesh(core_axis_name="c", subcore_axis_name="s")

_sc_gather_small = functools.partial(
    pl.kernel,
    mesh=_sc_mesh,
    out_type=[
        jax.ShapeDtypeStruct((N, KP), jnp.float32),
        jax.ShapeDtypeStruct((MH, KH), jnp.float32),
    ],
    scratch_types=[
        pltpu.VMEM((PW_P,), jnp.int32),
        pltpu.VMEM((PW_P, PER_D), jnp.float32),
        pltpu.VMEM((PW_H,), jnp.int32),
        pltpu.VMEM((PW_H, HH_D), jnp.float32),
        pltpu.SemaphoreType.DMA,
    ],
    compiler_params=pltpu.CompilerParams(use_tc_tiling_on_sc=False),
)(_sc_gather_small_body)

_sc_gather_act = functools.partial(
    pl.kernel,
    mesh=_sc_mesh,
    out_type=jax.ShapeDtypeStruct((MA, KA), jnp.float32),
    scratch_types=[
        pltpu.VMEM((CA,), jnp.int32),
        pltpu.VMEM((CA, ACT_D), jnp.float32),
        pltpu.SemaphoreType.DMA,
    ],
    compiler_params=pltpu.CompilerParams(use_tc_tiling_on_sc=False),
)(_sc_gather_act_body)


BN = 2048
NJ = N // BN
R_SMALL = 2 * H + 2


def _head_body(perg, hhg, p_w, p_b, h_w, h_b, sep_r, out):
    r = pl.program_id(0)
    is_per = r == 0
    is_hh = jnp.logical_and(r >= 2, lax.rem(r, 2) == 0)
    is_sep = jnp.logical_and(jnp.logical_not(is_per), jnp.logical_not(is_hh))

    @pl.when(is_per)
    def _():
        out[0] = jnp.dot(perg[...].astype(jnp.bfloat16),
                         p_w[...].astype(jnp.bfloat16),
                         preferred_element_type=jnp.float32) + p_b[...]

    @pl.when(is_hh)
    def _():
        out[0] = jnp.dot(hhg[0].astype(jnp.bfloat16),
                         h_w[...].astype(jnp.bfloat16),
                         preferred_element_type=jnp.float32) + h_b[...]

    @pl.when(is_sep)
    def _():
        out[0] = jnp.broadcast_to(sep_r[...], (BN, H2))


def _act_body(actg, a_w, a_b, alias, out):
    del alias
    out[0] = jnp.dot(actg[0].astype(jnp.bfloat16),
                     a_w[...].astype(jnp.bfloat16),
                     preferred_element_type=jnp.float32) + a_b[...]


def _project_head(per_g, hh_g, person_W, person_b, hh_W, hh_b, sep):
    zz = lambda r, j: (0, 0)
    hh_pred = lambda r: jnp.logical_and(r >= 2, lax.rem(r, 2) == 0)
    return pl.pallas_call(
        _head_body,
        grid=(R_SMALL, NJ),
        in_specs=[
            pl.BlockSpec((BN, KP), lambda r, j: (jnp.where(r == 0, j, 0), 0)),
            pl.BlockSpec((1, BN, KH),
                         lambda r, j: (jnp.where(hh_pred(r), (r - 2) // 2, 0),
                                       jnp.where(hh_pred(r), j, 0), 0)),
            pl.BlockSpec((KP, H2), zz),
            pl.BlockSpec((1, H2), zz),
            pl.BlockSpec((KH, H2), zz),
            pl.BlockSpec((1, H2), zz),
            pl.BlockSpec((1, H2), zz),
        ],
        out_specs=pl.BlockSpec((1, BN, H2), lambda r, j: (r, j, 0)),
        out_shape=jax.ShapeDtypeStruct((R_TOTAL, N, H2), jnp.float32),
    )(per_g, hh_g, person_W, person_b.reshape(1, H2), hh_W,
      hh_b.reshape(1, H2), sep.reshape(1, H2))


def _project_act(act_g, act_W, act_b, buf, half):
    zz = lambda r, j: (0, 0)
    base = R_SMALL + half * TH
    return pl.pallas_call(
        _act_body,
        grid=(TH, NJ),
        in_specs=[
            pl.BlockSpec((1, BN, KA), lambda r, j: (r, j, 0)),
            pl.BlockSpec((KA, H2), zz),
            pl.BlockSpec((1, H2), zz),
            pl.BlockSpec(memory_space=pl.ANY),
        ],
        out_specs=pl.BlockSpec((1, BN, H2), lambda r, j: (r + base, j, 0)),
        out_shape=jax.ShapeDtypeStruct((R_TOTAL, N, H2), jnp.float32),
        input_output_aliases={3: 0},
    )(act_g, act_W, act_b.reshape(1, H2), buf)


def kernel(activity_chain, target_person, household_members, act_tables,
           person_tables, hh_tables, act_W, act_b, person_W, person_b,
           hh_W, hh_b, sep):
    act_idx = [
        (activity_chain[h * TH:(h + 1) * TH].transpose(2, 0, 1)
         .reshape(ACT_F, MA)
         + jnp.arange(ACT_F, dtype=jnp.int32)[:, None] * ACT_V).reshape(-1)
        for h in range(NSPLIT)
    ]
    per_idx = (target_person.transpose(2, 0, 1).reshape(PER_F, N)
               + jnp.arange(PER_F, dtype=jnp.int32)[:, None] * PER_V).reshape(-1)
    hh_idx = (household_members.transpose(2, 0, 1).reshape(HH_F, MH)
              + jnp.arange(HH_F, dtype=jnp.int32)[:, None] * HH_V).reshape(-1)

    per_g, hh_g = _sc_gather_small(
        person_tables.reshape(PER_F * PER_V, PER_D), per_idx,
        hh_tables.reshape(HH_F * HH_V, HH_D), hh_idx)

    act_tab = act_tables.reshape(ACT_F * ACT_V, ACT_D)
    act_g = [_sc_gather_act(act_tab, act_idx[h]) for h in range(NSPLIT)]

    buf = _project_head(per_g, hh_g.reshape(H, N, KH),
                        person_W, person_b,
                        hh_W, hh_b, sep)

    for h in range(NSPLIT):
        buf = _project_act(act_g[h].reshape(TH, N, KA), act_W, act_b, buf, h)
    return buf


# speedup vs baseline: 1.3314x; 1.0130x over previous
import functools

import jax
import jax.numpy as jnp
from jax import lax
from jax.experimental import pallas as pl
from jax.experimental.pallas import tpu as pltpu
from jax.experimental.pallas import tpu_sc as plsc

H2 = 512
ACT_V, ACT_D, ACT_F = 100000, 64, 5
PER_V, PER_D, PER_F = 1000, 32, 23
HH_V, HH_D, HH_F = 1000, 32, 9
T, N, H = 50, 4096, 8
R_TOTAL = T + 2 * H + 2

KA, KP, KH = 320, 736, 288
DA, DP, DH = ACT_F * ACT_D, PER_F * PER_D, HH_F * HH_D

NC, NS = 2, 16
NW = NC * NS

NSPLIT = 2
TH = T // NSPLIT
MA = TH * N
MH = H * N
PW_A, PW_P, PW_H = MA // NW, N // NW, MH // NW
CA = 1600


def _sc_gather_small_body(per_tab, per_idx, hh_tab, hh_idx,
                          per_out, hh_out,
                          idx_p, rows_p, idx_h, rows_h, sem):
    wid = lax.axis_index("s") * NC + lax.axis_index("c")

    for f in range(PER_F):
        m = wid * PW_P
        pltpu.sync_copy(per_idx.at[pl.ds(f * N + m, PW_P)], idx_p)
        pltpu.async_copy(per_tab.at[idx_p], rows_p, sem).wait()
        pltpu.sync_copy(rows_p,
                        per_out.at[pl.ds(m, PW_P), pl.ds(f * PER_D, PER_D)])
    for f in range(HH_F):
        m = wid * PW_H
        pltpu.sync_copy(hh_idx.at[pl.ds(f * MH + m, PW_H)], idx_h)
        pltpu.async_copy(hh_tab.at[idx_h], rows_h, sem).wait()
        pltpu.sync_copy(rows_h,
                        hh_out.at[pl.ds(m, PW_H), pl.ds(f * HH_D, HH_D)])


def _sc_gather_act_body(act_tab, act_idx, act_out, idx_a, rows_a, sem):
    wid = lax.axis_index("s") * NC + lax.axis_index("c")

    for f in range(ACT_F):
        base = wid * PW_A

        def body_a(i, carry, f=f, base=base):
            m = base + i * CA
            pltpu.sync_copy(act_idx.at[pl.ds(f * MA + m, CA)], idx_a)
            pltpu.async_copy(act_tab.at[idx_a], rows_a, sem).wait()
            pltpu.sync_copy(rows_a,
                            act_out.at[pl.ds(m, CA), pl.ds(f * ACT_D, ACT_D)])
            return carry

        lax.fori_loop(0, PW_A // CA, body_a, 0)


_sc_mesh = plsc.VectorSubcoreMesh(core_axis_name="c", subcore_axis_name="s")

_sc_gather_small = functools.partial(
    pl.kernel,
    mesh=_sc_mesh,
    out_type=[
        jax.ShapeDtypeStruct((N, KP), jnp.float32),
        jax.ShapeDtypeStruct((MH, KH), jnp.float32),
    ],
    scratch_types=[
        pltpu.VMEM((PW_P,), jnp.int32),
        pltpu.VMEM((PW_P, PER_D), jnp.float32),
        pltpu.VMEM((PW_H,), jnp.int32),
        pltpu.VMEM((PW_H, HH_D), jnp.float32),
        pltpu.SemaphoreType.DMA,
    ],
    compiler_params=pltpu.CompilerParams(use_tc_tiling_on_sc=False),
)(_sc_gather_small_body)

_sc_gather_act = functools.partial(
    pl.kernel,
    mesh=_sc_mesh,
    out_type=jax.ShapeDtypeStruct((MA, KA), jnp.float32),
    scratch_types=[
        pltpu.VMEM((CA,), jnp.int32),
        pltpu.VMEM((CA, ACT_D), jnp.float32),
        pltpu.SemaphoreType.DMA,
    ],
    compiler_params=pltpu.CompilerParams(use_tc_tiling_on_sc=False),
)(_sc_gather_act_body)


BN = 2048
NJ = N // BN
BNA = 4096
NJA = N // BNA
R_SMALL = 2 * H + 2


def _head_body(perg, hhg, p_w, p_b, h_w, h_b, sep_r, out):
    r = pl.program_id(0)
    is_per = r == 0
    is_hh = jnp.logical_and(r >= 2, lax.rem(r, 2) == 0)
    is_sep = jnp.logical_and(jnp.logical_not(is_per), jnp.logical_not(is_hh))

    @pl.when(is_per)
    def _():
        out[0] = jnp.dot(perg[...].astype(jnp.bfloat16),
                         p_w[...].astype(jnp.bfloat16),
                         preferred_element_type=jnp.float32) + p_b[...]

    @pl.when(is_hh)
    def _():
        out[0] = jnp.dot(hhg[0].astype(jnp.bfloat16),
                         h_w[...].astype(jnp.bfloat16),
                         preferred_element_type=jnp.float32) + h_b[...]

    @pl.when(is_sep)
    def _():
        out[0] = jnp.broadcast_to(sep_r[...], (BN, H2))


def _act_body(actg, a_w, a_b, alias, out):
    del alias
    out[0] = jnp.dot(actg[0].astype(jnp.bfloat16),
                     a_w[...].astype(jnp.bfloat16),
                     preferred_element_type=jnp.float32) + a_b[...]


def _project_head(per_g, hh_g, person_W, person_b, hh_W, hh_b, sep):
    zz = lambda r, j: (0, 0)
    hh_pred = lambda r: jnp.logical_and(r >= 2, lax.rem(r, 2) == 0)
    return pl.pallas_call(
        _head_body,
        grid=(R_SMALL, NJ),
        in_specs=[
            pl.BlockSpec((BN, KP), lambda r, j: (jnp.where(r == 0, j, 0), 0)),
            pl.BlockSpec((1, BN, KH),
                         lambda r, j: (jnp.where(hh_pred(r), (r - 2) // 2, 0),
                                       jnp.where(hh_pred(r), j, 0), 0)),
            pl.BlockSpec((KP, H2), zz),
            pl.BlockSpec((1, H2), zz),
            pl.BlockSpec((KH, H2), zz),
            pl.BlockSpec((1, H2), zz),
            pl.BlockSpec((1, H2), zz),
        ],
        out_specs=pl.BlockSpec((1, BN, H2), lambda r, j: (r, j, 0)),
        out_shape=jax.ShapeDtypeStruct((R_TOTAL, N, H2), jnp.float32),
    )(per_g, hh_g, person_W, person_b.reshape(1, H2), hh_W,
      hh_b.reshape(1, H2), sep.reshape(1, H2))


def _project_act(act_g, act_W, act_b, buf, half):
    zz = lambda r, j: (0, 0)
    base = R_SMALL + half * TH
    return pl.pallas_call(
        _act_body,
        grid=(TH, NJA),
        in_specs=[
            pl.BlockSpec((1, BNA, KA), lambda r, j: (r, j, 0)),
            pl.BlockSpec((KA, H2), zz),
            pl.BlockSpec((1, H2), zz),
            pl.BlockSpec(memory_space=pl.ANY),
        ],
        out_specs=pl.BlockSpec((1, BNA, H2), lambda r, j: (r + base, j, 0)),
        out_shape=jax.ShapeDtypeStruct((R_TOTAL, N, H2), jnp.float32),
        input_output_aliases={3: 0},
    )(act_g, act_W, act_b.reshape(1, H2), buf)


def kernel(activity_chain, target_person, household_members, act_tables,
           person_tables, hh_tables, act_W, act_b, person_W, person_b,
           hh_W, hh_b, sep):
    act_idx = [
        (activity_chain[h * TH:(h + 1) * TH].transpose(2, 0, 1)
         .reshape(ACT_F, MA)
         + jnp.arange(ACT_F, dtype=jnp.int32)[:, None] * ACT_V).reshape(-1)
        for h in range(NSPLIT)
    ]
    per_idx = (target_person.transpose(2, 0, 1).reshape(PER_F, N)
               + jnp.arange(PER_F, dtype=jnp.int32)[:, None] * PER_V).reshape(-1)
    hh_idx = (household_members.transpose(2, 0, 1).reshape(HH_F, MH)
              + jnp.arange(HH_F, dtype=jnp.int32)[:, None] * HH_V).reshape(-1)

    per_g, hh_g = _sc_gather_small(
        person_tables.reshape(PER_F * PER_V, PER_D), per_idx,
        hh_tables.reshape(HH_F * HH_V, HH_D), hh_idx)

    act_tab = act_tables.reshape(ACT_F * ACT_V, ACT_D)
    act_g = [_sc_gather_act(act_tab, act_idx[h]) for h in range(NSPLIT)]

    buf = _project_head(per_g, hh_g.reshape(H, N, KH),
                        person_W, person_b,
                        hh_W, hh_b, sep)

    for h in range(NSPLIT):
        buf = _project_act(act_g[h].reshape(TH, N, KA), act_W, act_b, buf, h)
    return buf
